# baseline (speedup 1.0000x reference)
import jax
import jax.numpy as jnp
from jax.experimental import pallas as pl
from jax.experimental.pallas import tpu as pltpu

_BN = 1024


def _split3_bf16(v):
    hi = v.astype(jnp.bfloat16)
    r = v - hi.astype(jnp.float32)
    mid = r.astype(jnp.bfloat16)
    lo = (r - mid.astype(jnp.float32)).astype(jnp.bfloat16)
    return hi, mid, lo


def _transpose_body(x2_ref, x2t_ref):
    x2t_ref[0] = x2_ref[0].T


def _chamfer_body(x1_ref, x2t_ref, d1_ref, d2_ref):
    i = pl.program_id(1)

    x1 = x1_ref[0]
    x2t = x2t_ref[0]

    n1 = jnp.sum(x1 * x1, axis=1, keepdims=True)
    n2 = jnp.sum(x2t * x2t, axis=0, keepdims=True)

    n1h, n1m, n1l = _split3_bf16(n1)
    n2h, n2m, n2l = _split3_bf16(n2)
    bn = x1.shape[0]
    m = x2t.shape[1]

    lhs = jnp.concatenate(
        [(-2.0 * x1).astype(jnp.bfloat16),
         jnp.ones((bn, 3), jnp.bfloat16), n1h, n1m, n1l], axis=1)
    rhs = jnp.concatenate(
        [x2t.astype(jnp.bfloat16), n2h, n2m, n2l,
         jnp.ones((3, m), jnp.bfloat16)], axis=0)

    d2 = jnp.dot(lhs, rhs, preferred_element_type=jnp.float32)

    part = d2[:, 0:128]
    for k in range(1, m // 128):
        part = jnp.minimum(part, d2[:, k * 128:(k + 1) * 128])
    d1_ref[0, 0, :] = jnp.maximum(jnp.min(part.T, axis=0), 0.0)

    col_min = jnp.maximum(jnp.min(d2, axis=0, keepdims=True), 0.0)[None]

    @pl.when(i == 0)
    def _():
        d2_ref[...] = col_min

    @pl.when(i > 0)
    def _():
        d2_ref[...] = jnp.minimum(d2_ref[...], col_min)


def kernel(xyz1, xyz2):
    xyz1 = xyz1.astype(jnp.float32)
    xyz2 = xyz2.astype(jnp.float32)
    B, N, _ = xyz1.shape
    _, M, _ = xyz2.shape

    x2t = pl.pallas_call(
        _transpose_body,
        grid=(B,),
        in_specs=[pl.BlockSpec((1, M, 3), lambda b: (b, 0, 0))],
        out_specs=pl.BlockSpec((1, 3, M), lambda b: (b, 0, 0)),
        out_shape=jax.ShapeDtypeStruct((B, 3, M), jnp.float32),
    )(xyz2)

    grid = (B, N // _BN)
    dist1, dist2 = pl.pallas_call(
        _chamfer_body,
        grid=grid,
        in_specs=[
            pl.BlockSpec((1, _BN, 3), lambda b, i: (b, i, 0)),
            pl.BlockSpec((1, 3, M), lambda b, i: (b, 0, 0)),
        ],
        out_specs=[
            pl.BlockSpec((1, 1, _BN), lambda b, i: (b, 0, i)),
            pl.BlockSpec((1, 1, M), lambda b, i: (b, 0, 0)),
        ],
        out_shape=[
            jax.ShapeDtypeStruct((B, 1, N), jnp.float32),
            jax.ShapeDtypeStruct((B, 1, M), jnp.float32),
        ],
        compiler_params=pltpu.CompilerParams(
            dimension_semantics=("parallel", "arbitrary"),
        ),
    )(xyz1, x2t)
    return (dist1[:, 0, :], dist2[:, 0, :])


# speedup vs baseline: 1.1702x; 1.1702x over previous
import jax
import jax.numpy as jnp
from jax.experimental import pallas as pl
from jax.experimental.pallas import tpu as pltpu

_BN = 1024


def _split3_bf16(v):
    hi = v.astype(jnp.bfloat16)
    r = v - hi.astype(jnp.float32)
    mid = r.astype(jnp.bfloat16)
    lo = (r - mid.astype(jnp.float32)).astype(jnp.bfloat16)
    return hi, mid, lo


def _chamfer_body(x1_ref, x2t_ref, d1_ref, d2_ref):
    i = pl.program_id(1)

    x1 = x1_ref[0]
    x2t = x2t_ref[0]

    n1 = jnp.sum(x1 * x1, axis=1, keepdims=True)
    n2 = jnp.sum(x2t * x2t, axis=0, keepdims=True)

    n1h, n1m, n1l = _split3_bf16(n1)
    n2h, n2m, n2l = _split3_bf16(n2)
    bn = x1.shape[0]
    m = x2t.shape[1]

    lhs = jnp.concatenate(
        [(-2.0 * x1).astype(jnp.bfloat16),
         jnp.ones((bn, 3), jnp.bfloat16), n1h, n1m, n1l], axis=1)
    rhs = jnp.concatenate(
        [x2t.astype(jnp.bfloat16), n2h, n2m, n2l,
         jnp.ones((3, m), jnp.bfloat16)], axis=0)

    d2 = jnp.dot(lhs, rhs, preferred_element_type=jnp.float32)

    slabs = [d2[:, k * 128:(k + 1) * 128] for k in range(m // 128)]
    while len(slabs) > 1:
        half = len(slabs) // 2
        slabs = [jnp.minimum(slabs[j], slabs[half + j]) for j in range(half)]
    d1_ref[0, 0, :] = jnp.maximum(jnp.min(slabs[0].T, axis=0), 0.0)

    col_min = jnp.maximum(jnp.min(d2, axis=0, keepdims=True), 0.0)[None]

    @pl.when(i == 0)
    def _():
        d2_ref[...] = col_min

    @pl.when(i > 0)
    def _():
        d2_ref[...] = jnp.minimum(d2_ref[...], col_min)


def kernel(xyz1, xyz2):
    xyz1 = xyz1.astype(jnp.float32)
    xyz2 = xyz2.astype(jnp.float32)
    B, N, _ = xyz1.shape
    _, M, _ = xyz2.shape

    x2t = jnp.swapaxes(xyz2, 1, 2)

    grid = (B, N // _BN)
    dist1, dist2 = pl.pallas_call(
        _chamfer_body,
        grid=grid,
        in_specs=[
            pl.BlockSpec((1, _BN, 3), lambda b, i: (b, i, 0)),
            pl.BlockSpec((1, 3, M), lambda b, i: (b, 0, 0)),
        ],
        out_specs=[
            pl.BlockSpec((1, 1, _BN), lambda b, i: (b, 0, i)),
            pl.BlockSpec((1, 1, M), lambda b, i: (b, 0, 0)),
        ],
        out_shape=[
            jax.ShapeDtypeStruct((B, 1, N), jnp.float32),
            jax.ShapeDtypeStruct((B, 1, M), jnp.float32),
        ],
        compiler_params=pltpu.CompilerParams(
            dimension_semantics=("parallel", "arbitrary"),
        ),
    )(xyz1, x2t)
    return (dist1[:, 0, :], dist2[:, 0, :])


# speedup vs baseline: 1.1715x; 1.0011x over previous
import jax
import jax.numpy as jnp
from jax.experimental import pallas as pl
from jax.experimental.pallas import tpu as pltpu

_BN = 1024


def _split3_bf16(v):
    hi = v.astype(jnp.bfloat16)
    r = v - hi.astype(jnp.float32)
    mid = r.astype(jnp.bfloat16)
    lo = (r - mid.astype(jnp.float32)).astype(jnp.bfloat16)
    return hi, mid, lo


def _chamfer_body(x1_ref, x2t_ref, d1_ref, d2_ref):
    i = pl.program_id(1)

    x1 = x1_ref[0]
    x2t = x2t_ref[0]

    n1 = jnp.sum(x1 * x1, axis=1, keepdims=True)
    n2 = jnp.sum(x2t * x2t, axis=0, keepdims=True)

    n1h, n1m, n1l = _split3_bf16(n1)
    n2h, n2m, n2l = _split3_bf16(n2)
    bn = x1.shape[0]
    m = x2t.shape[1]

    lhs = jnp.concatenate(
        [(-2.0 * x1).astype(jnp.bfloat16),
         jnp.ones((bn, 3), jnp.bfloat16), n1h, n1m, n1l], axis=1)
    rhs = jnp.concatenate(
        [x2t.astype(jnp.bfloat16), n2h, n2m, n2l,
         jnp.ones((3, m), jnp.bfloat16)], axis=0)

    ch = 1024
    parts = []
    cols = []
    for c in range(m // ch):
        dc = jnp.dot(lhs, rhs[:, c * ch:(c + 1) * ch],
                     preferred_element_type=jnp.float32)
        slabs = [dc[:, k * 128:(k + 1) * 128] for k in range(ch // 128)]
        while len(slabs) > 1:
            half = len(slabs) // 2
            slabs = [jnp.minimum(slabs[j], slabs[half + j])
                     for j in range(half)]
        parts.append(slabs[0])
        cols.append(jnp.min(dc, axis=0, keepdims=True))

    while len(parts) > 1:
        half = len(parts) // 2
        parts = [jnp.minimum(parts[j], parts[half + j]) for j in range(half)]
    d1_ref[0, 0, :] = jnp.maximum(jnp.min(parts[0].T, axis=0), 0.0)

    col_min = jnp.maximum(jnp.concatenate(cols, axis=1), 0.0)[None]

    @pl.when(i == 0)
    def _():
        d2_ref[...] = col_min

    @pl.when(i > 0)
    def _():
        d2_ref[...] = jnp.minimum(d2_ref[...], col_min)


def kernel(xyz1, xyz2):
    xyz1 = xyz1.astype(jnp.float32)
    xyz2 = xyz2.astype(jnp.float32)
    B, N, _ = xyz1.shape
    _, M, _ = xyz2.shape

    x2t = jnp.swapaxes(xyz2, 1, 2)

    grid = (B, N // _BN)
    dist1, dist2 = pl.pallas_call(
        _chamfer_body,
        grid=grid,
        in_specs=[
            pl.BlockSpec((1, _BN, 3), lambda b, i: (b, i, 0)),
            pl.BlockSpec((1, 3, M), lambda b, i: (b, 0, 0)),
        ],
        out_specs=[
            pl.BlockSpec((1, 1, _BN), lambda b, i: (b, 0, i)),
            pl.BlockSpec((1, 1, M), lambda b, i: (b, 0, 0)),
        ],
        out_shape=[
            jax.ShapeDtypeStruct((B, 1, N), jnp.float32),
            jax.ShapeDtypeStruct((B, 1, M), jnp.float32),
        ],
        compiler_params=pltpu.CompilerParams(
            dimension_semantics=("parallel", "arbitrary"),
        ),
    )(xyz1, x2t)
    return (dist1[:, 0, :], dist2[:, 0, :])


# speedup vs baseline: 1.2331x; 1.0526x over previous
import jax
import jax.numpy as jnp
from jax.experimental import pallas as pl
from jax.experimental.pallas import tpu as pltpu

_BN = 2048


def _split3_bf16(v):
    hi = v.astype(jnp.bfloat16)
    r = v - hi.astype(jnp.float32)
    mid = r.astype(jnp.bfloat16)
    lo = (r - mid.astype(jnp.float32)).astype(jnp.bfloat16)
    return hi, mid, lo


def _chamfer_body(x1_ref, x2t_ref, d1_ref, d2_ref):
    i = pl.program_id(1)

    x1 = x1_ref[0]
    x2t = x2t_ref[0]

    n1 = jnp.sum(x1 * x1, axis=1, keepdims=True)
    n2 = jnp.sum(x2t * x2t, axis=0, keepdims=True)

    n1h, n1m, n1l = _split3_bf16(n1)
    n2h, n2m, n2l = _split3_bf16(n2)
    bn = x1.shape[0]
    m = x2t.shape[1]

    lhs = jnp.concatenate(
        [(-2.0 * x1).astype(jnp.bfloat16),
         jnp.ones((bn, 3), jnp.bfloat16), n1h, n1m, n1l], axis=1)
    rhs = jnp.concatenate(
        [x2t.astype(jnp.bfloat16), n2h, n2m, n2l,
         jnp.ones((3, m), jnp.bfloat16)], axis=0)

    ch = 1024
    parts = []
    cols = []
    for c in range(m // ch):
        dc = jnp.dot(lhs, rhs[:, c * ch:(c + 1) * ch],
                     preferred_element_type=jnp.float32)
        slabs = [dc[:, k * 128:(k + 1) * 128] for k in range(ch // 128)]
        while len(slabs) > 1:
            half = len(slabs) // 2
            slabs = [jnp.minimum(slabs[j], slabs[half + j])
                     for j in range(half)]
        parts.append(slabs[0])
        cols.append(jnp.min(dc, axis=0, keepdims=True))

    while len(parts) > 1:
        half = len(parts) // 2
        parts = [jnp.minimum(parts[j], parts[half + j]) for j in range(half)]
    d1_ref[0, 0, :] = jnp.maximum(jnp.min(parts[0].T, axis=0), 0.0)

    col_min = jnp.maximum(jnp.concatenate(cols, axis=1), 0.0)[None]

    @pl.when(i == 0)
    def _():
        d2_ref[...] = col_min

    @pl.when(i > 0)
    def _():
        d2_ref[...] = jnp.minimum(d2_ref[...], col_min)


def kernel(xyz1, xyz2):
    xyz1 = xyz1.astype(jnp.float32)
    xyz2 = xyz2.astype(jnp.float32)
    B, N, _ = xyz1.shape
    _, M, _ = xyz2.shape

    x2t = jnp.swapaxes(xyz2, 1, 2)

    grid = (B, N // _BN)
    dist1, dist2 = pl.pallas_call(
        _chamfer_body,
        grid=grid,
        in_specs=[
            pl.BlockSpec((1, _BN, 3), lambda b, i: (b, i, 0)),
            pl.BlockSpec((1, 3, M), lambda b, i: (b, 0, 0)),
        ],
        out_specs=[
            pl.BlockSpec((1, 1, _BN), lambda b, i: (b, 0, i)),
            pl.BlockSpec((1, 1, M), lambda b, i: (b, 0, 0)),
        ],
        out_shape=[
            jax.ShapeDtypeStruct((B, 1, N), jnp.float32),
            jax.ShapeDtypeStruct((B, 1, M), jnp.float32),
        ],
        compiler_params=pltpu.CompilerParams(
            dimension_semantics=("parallel", "arbitrary"),
        ),
    )(xyz1, x2t)
    return (dist1[:, 0, :], dist2[:, 0, :])


# speedup vs baseline: 1.2593x; 1.0212x over previous
import jax
import jax.numpy as jnp
from jax.experimental import pallas as pl
from jax.experimental.pallas import tpu as pltpu

_BN = 4096


def _split3_bf16(v):
    hi = v.astype(jnp.bfloat16)
    r = v - hi.astype(jnp.float32)
    mid = r.astype(jnp.bfloat16)
    lo = (r - mid.astype(jnp.float32)).astype(jnp.bfloat16)
    return hi, mid, lo


def _chamfer_body(x1_ref, x2t_ref, d1_ref, d2_ref):
    i = pl.program_id(1)

    x1 = x1_ref[0]
    x2t = x2t_ref[0]

    n1 = jnp.sum(x1 * x1, axis=1, keepdims=True)
    n2 = jnp.sum(x2t * x2t, axis=0, keepdims=True)

    n1h, n1m, n1l = _split3_bf16(n1)
    n2h, n2m, n2l = _split3_bf16(n2)
    bn = x1.shape[0]
    m = x2t.shape[1]

    lhs = jnp.concatenate(
        [(-2.0 * x1).astype(jnp.bfloat16),
         jnp.ones((bn, 3), jnp.bfloat16), n1h, n1m, n1l], axis=1)
    rhs = jnp.concatenate(
        [x2t.astype(jnp.bfloat16), n2h, n2m, n2l,
         jnp.ones((3, m), jnp.bfloat16)], axis=0)

    ch = 1024
    parts = []
    cols = []
    for c in range(m // ch):
        dc = jnp.dot(lhs, rhs[:, c * ch:(c + 1) * ch],
                     preferred_element_type=jnp.float32)
        slabs = [dc[:, k * 128:(k + 1) * 128] for k in range(ch // 128)]
        while len(slabs) > 1:
            half = len(slabs) // 2
            slabs = [jnp.minimum(slabs[j], slabs[half + j])
                     for j in range(half)]
        parts.append(slabs[0])
        cols.append(jnp.min(dc, axis=0, keepdims=True))

    while len(parts) > 1:
        half = len(parts) // 2
        parts = [jnp.minimum(parts[j], parts[half + j]) for j in range(half)]
    d1_ref[0, 0, :] = jnp.maximum(jnp.min(parts[0].T, axis=0), 0.0)

    col_min = jnp.maximum(jnp.concatenate(cols, axis=1), 0.0)[None]

    @pl.when(i == 0)
    def _():
        d2_ref[...] = col_min

    @pl.when(i > 0)
    def _():
        d2_ref[...] = jnp.minimum(d2_ref[...], col_min)


def kernel(xyz1, xyz2):
    xyz1 = xyz1.astype(jnp.float32)
    xyz2 = xyz2.astype(jnp.float32)
    B, N, _ = xyz1.shape
    _, M, _ = xyz2.shape

    x2t = jnp.swapaxes(xyz2, 1, 2)

    grid = (B, N // _BN)
    dist1, dist2 = pl.pallas_call(
        _chamfer_body,
        grid=grid,
        in_specs=[
            pl.BlockSpec((1, _BN, 3), lambda b, i: (b, i, 0)),
            pl.BlockSpec((1, 3, M), lambda b, i: (b, 0, 0)),
        ],
        out_specs=[
            pl.BlockSpec((1, 1, _BN), lambda b, i: (b, 0, i)),
            pl.BlockSpec((1, 1, M), lambda b, i: (b, 0, 0)),
        ],
        out_shape=[
            jax.ShapeDtypeStruct((B, 1, N), jnp.float32),
            jax.ShapeDtypeStruct((B, 1, M), jnp.float32),
        ],
        compiler_params=pltpu.CompilerParams(
            dimension_semantics=("parallel", "arbitrary"),
        ),
    )(xyz1, x2t)
    return (dist1[:, 0, :], dist2[:, 0, :])
